# baseline (speedup 1.0000x reference)
import functools

import jax
import jax.numpy as jnp
from jax import lax
from jax.experimental import pallas as pl
from jax.experimental.pallas import tpu as pltpu
from jax.experimental.pallas import tpu_sc as plsc

F = 26
V = 100000
D = 32
B = 16384
CD = 64
XR = CD // D
OR = F + XR

NC = 2
NS = 16
NW = NC * NS
BW = B // NW
CB = 64
NCHUNK = BW // CB
ROWS = CB * F
NT = ROWS // 128
NIV = ROWS // 16


def _body(x2, vals, tab, out, vals_v, gidx_v, oidx_v, xidx_v, rows_v, x_v,
          patg_v, pato_v, patx_v, sg0, sg1, ss0, ss1):
    wid = lax.axis_index("s") * NC + lax.axis_index("c")
    sem_g = (sg0, sg1)
    sem_s = (ss0, ss1)

    idx16 = lax.iota(jnp.int32, 16)

    def _splat(c):
        return jnp.full((16,), c, jnp.int32)

    for j in range(13):
        q = idx16 + j * 16
        f = lax.rem(q, _splat(F))
        patg_v[j, :] = f * V
        pato_v[j, :] = lax.div(q, _splat(F)) * OR + f
    patx_v[:] = lax.div(idx16, _splat(XR)) * OR + F + lax.rem(idx16, _splat(XR))

    def load_and_index(c, p):
        r0 = wid * BW + c * CB
        pltpu.sync_copy(vals.at[pl.ds(r0 * F, ROWS)], vals_v.at[p])
        pltpu.sync_copy(x2.at[pl.ds(r0 * XR, CB * XR)], x_v.at[p])
        base_out = r0 * OR

        def ivec(i, carry):
            j = lax.rem(i, 13)
            g = lax.div(i, 13)
            row = lax.div(i, 8)
            col = lax.rem(i, 8) * 16
            vv = vals_v[p, pl.ds(i * 16, 16)]
            gidx_v[p, row, pl.ds(col, 16)] = vv + patg_v[j, :]
            goff = base_out + g * (208 // F * OR)
            oidx_v[p, row, pl.ds(col, 16)] = pato_v[j, :] + goff
            return carry

        lax.fori_loop(0, NIV, ivec, 0)
        for k in range(CB * XR // 16):
            xidx_v[p, pl.ds(k * 16, 16)] = patx_v[:] + (base_out + k * 8 * OR)

    def fire_gathers(p):
        return [
            pltpu.async_copy(
                tab.at[gidx_v.at[p, t]],
                rows_v.at[p, pl.ds(t * 128, 128)],
                sem_g[p],
            )
            for t in range(NT)
        ]

    def fire_scatters(p):
        cps = [
            pltpu.async_copy(
                rows_v.at[p, pl.ds(t * 128, 128)],
                out.at[oidx_v.at[p, t]],
                sem_s[p],
            )
            for t in range(NT)
        ]
        cps.append(pltpu.async_copy(x_v.at[p], out.at[xidx_v.at[p]], sem_s[p]))
        return cps

    g_cp = {}
    s_cp = {}
    for c in range(NCHUNK):
        p = c & 1
        if c >= 2:
            for cp in s_cp[c - 2]:
                cp.wait()
        load_and_index(c, p)
        g_cp[c] = fire_gathers(p)
        if c >= 1:
            for cp in g_cp[c - 1]:
                cp.wait()
            s_cp[c - 1] = fire_scatters(1 - p)
    for cp in s_cp[NCHUNK - 2]:
        cp.wait()
    last = NCHUNK - 1
    for cp in g_cp[last]:
        cp.wait()
    for cp in fire_scatters(last & 1):
        cp.wait()


@jax.jit
def _run(x2, vals, tab):
    mesh = plsc.VectorSubcoreMesh(core_axis_name="c", subcore_axis_name="s")
    kern = functools.partial(
        pl.kernel,
        out_type=jax.ShapeDtypeStruct((B * OR, D), jnp.float32),
        mesh=mesh,
        compiler_params=pltpu.CompilerParams(use_tc_tiling_on_sc=False),
        scratch_types=[
            pltpu.VMEM((2, ROWS), jnp.int32),
            pltpu.VMEM((2, NT, 128), jnp.int32),
            pltpu.VMEM((2, NT, 128), jnp.int32),
            pltpu.VMEM((2, 128), jnp.int32),
            pltpu.VMEM((2, ROWS, D), jnp.float32),
            pltpu.VMEM((2, CB * XR, D), jnp.float32),
            pltpu.VMEM((13, 16), jnp.int32),
            pltpu.VMEM((13, 16), jnp.int32),
            pltpu.VMEM((16,), jnp.int32),
            pltpu.SemaphoreType.DMA,
            pltpu.SemaphoreType.DMA,
            pltpu.SemaphoreType.DMA,
            pltpu.SemaphoreType.DMA,
        ],
    )(_body)
    return kern(x2, vals, tab)


def kernel(x, values, tables):
    x2 = x.reshape(B * XR, D)
    vals = values.reshape(B * F)
    tab = tables.reshape(F * V, D)
    out = _run(x2, vals, tab)
    return out.reshape(B, F * D + CD)
